# baseline (speedup 1.0000x reference)
import numpy as np
import jax
import jax.numpy as jnp
from jax.experimental import pallas as pl
from jax.experimental.pallas import tpu as pltpu

N = 4096
D = 3
TR = 1024
NPAIR = 16
NT = N // TR
FLAT = N * D

_POS_LO = np.float32(0.15) * np.float32(FLAT - 1)
_POS_HI = np.float32(0.85) * np.float32(FLAT - 1)
RANK_LO = int(np.floor(_POS_LO))
RANK_HI = int(np.floor(_POS_HI))
FRAC_LO = np.float32(_POS_LO - np.floor(_POS_LO))
FRAC_HI = np.float32(_POS_HI - np.floor(_POS_HI))

INT_MIN = np.int32(-2**31)
INT_MAX = np.int32(2**31 - 1)


def _residual_kernel(s1_ref, s2t_ref, s2n_ref, out_ref):
    s1 = s1_ref[0]
    s2t = s2t_ref[0]
    s2n = s2n_ref[0]
    b0 = s2t[0:1, :]
    b1 = s2t[1:2, :]
    b2 = s2t[2:3, :]
    a0 = s1[:, 0:1]
    a1 = s1[:, 1:2]
    a2 = s1[:, 2:3]
    s2sq = (b0 * b0 + b1 * b1) + b2 * b2
    hi = s2sq.astype(jnp.bfloat16)
    r1 = s2sq - hi.astype(jnp.float32)
    mid = r1.astype(jnp.bfloat16)
    lo = (r1 - mid.astype(jnp.float32)).astype(jnp.bfloat16)
    w = jnp.concatenate(
        [(-(s2t + s2t)).astype(jnp.bfloat16), hi, mid, lo], axis=0)
    ones = jnp.ones((TR, 3), dtype=jnp.bfloat16)
    s1aug = jnp.concatenate([s1.astype(jnp.bfloat16), ones], axis=1)
    scores = jnp.dot(s1aug, w, preferred_element_type=jnp.float32)
    mins = jnp.min(scores, axis=1, keepdims=True)
    onehotb = jnp.where(scores == mins, jnp.float32(1),
                        jnp.float32(0)).astype(jnp.bfloat16)
    ghi = s2n.astype(jnp.bfloat16)
    gr1 = s2n - ghi.astype(jnp.float32)
    gmid = gr1.astype(jnp.bfloat16)
    glo = (gr1 - gmid.astype(jnp.float32)).astype(jnp.bfloat16)
    gones = jnp.ones((N, 1), dtype=jnp.bfloat16)
    w2 = jnp.concatenate([ghi, gmid, glo, gones], axis=1)
    g = jnp.dot(onehotb, w2, preferred_element_type=jnp.float32)
    cnt = g[:, 9:10]
    c0 = ((g[:, 0:1] + g[:, 3:4]) + g[:, 6:7]) / cnt
    c1 = ((g[:, 1:2] + g[:, 4:5]) + g[:, 7:8]) / cnt
    c2 = ((g[:, 2:3] + g[:, 5:6]) + g[:, 8:9]) / cnt
    out_ref[0] = jnp.concatenate([a0 - c0, a1 - c1, a2 - c2], axis=1)


def _to_ordered(i):
    neg = jnp.bitwise_xor(jnp.bitwise_not(i), INT_MIN)
    return jnp.where(i >= 0, i, neg)


def _from_ordered(k):
    i = jnp.where(k >= 0, k, jnp.bitwise_not(jnp.bitwise_xor(k, INT_MIN)))
    return jax.lax.bitcast_convert_type(i, jnp.float32)


def _count_lt(keys, t):
    return jnp.sum((keys < t).astype(jnp.int32), axis=(1, 2), keepdims=True)


def _order_stat(keys, rank):
    def body(b, a):
        bit = jnp.left_shift(jnp.int32(1), 31 - b)
        t = a + bit
        cnt = _count_lt(keys, t)
        return jnp.where(cnt <= rank, t, a)
    a0 = jnp.full((keys.shape[0], 1, 1), INT_MIN, dtype=jnp.int32)
    return jax.lax.fori_loop(0, 32, body, a0)


def _next_order_stat(keys, v, rank):
    c_le = jnp.sum((keys <= v).astype(jnp.int32), axis=(1, 2), keepdims=True)
    above = jnp.where(keys > v, keys, INT_MAX)
    nxt = jnp.min(above, axis=(1, 2), keepdims=True)
    return jnp.where(c_le >= rank + 2, v, nxt)


def _masked_std1(r, mask):
    zero = jnp.float32(0.0)
    nm = jnp.sum(jnp.where(mask, jnp.float32(1.0), zero), axis=(1, 2),
                 keepdims=True)
    tot = jnp.sum(jnp.where(mask, r, zero), axis=(1, 2), keepdims=True)
    mean = tot / nm
    sq = jnp.sum(jnp.where(mask, (r - mean) ** 2, zero), axis=(1, 2),
                 keepdims=True)
    return jnp.sqrt(sq / (nm - 1.0)), nm


def _stats_kernel(r_ref, out_ref):
    r = r_ref[0]
    i = jax.lax.bitcast_convert_type(r, jnp.int32)
    keys = _to_ordered(i)

    v_lo0 = _order_stat(keys, RANK_LO)
    v_lo1 = _next_order_stat(keys, v_lo0, RANK_LO)
    v_hi0 = _order_stat(keys, RANK_HI)
    v_hi1 = _next_order_stat(keys, v_hi0, RANK_HI)

    q0 = _from_ordered(v_lo0) * (1.0 - FRAC_LO) + _from_ordered(v_lo1) * FRAC_LO
    q1 = _from_ordered(v_hi0) * (1.0 - FRAC_HI) + _from_ordered(v_hi1) * FRAC_HI

    mask = (r < q0) | (r > q1)
    std_m, nm = _masked_std1(r, mask)
    all_false = nm == 0.0

    nf = jnp.float32(FLAT)
    mean_a = jnp.sum(r, axis=(1, 2), keepdims=True) / nf
    ss = jnp.sum((r - mean_a) ** 2, axis=(1, 2), keepdims=True)
    std_a = jnp.sqrt(ss / (nf - 1.0))
    mask2 = jnp.abs(r - mean_a) > std_a
    std_m2, nm2 = _masked_std1(r, mask2)
    fb = jnp.where(nm2 == 0.0, std_a, std_m2)

    stds = jnp.where(all_false, fb, std_m)
    out_ref[0] = jnp.broadcast_to(stds.reshape(8, 1), (8, 128))


def kernel(x, y):
    s1 = jnp.concatenate([x, y], axis=0)
    s2 = jnp.concatenate([y, x], axis=0)
    s2t = s2.transpose(0, 2, 1)

    resid = pl.pallas_call(
        _residual_kernel,
        grid=(NPAIR, NT),
        in_specs=[
            pl.BlockSpec((1, TR, D), lambda b, t: (b, t, 0)),
            pl.BlockSpec((1, D, N), lambda b, t: (b, 0, 0)),
            pl.BlockSpec((1, N, D), lambda b, t: (b, 0, 0)),
        ],
        out_specs=pl.BlockSpec((1, TR, D), lambda b, t: (b, t, 0)),
        out_shape=jax.ShapeDtypeStruct((NPAIR, N, D), jnp.float32),
        compiler_params=pltpu.CompilerParams(
            dimension_semantics=("parallel", "arbitrary")),
    )(s1, s2t, s2)

    r = resid.reshape(2, NPAIR // 2, FLAT // 128, 128)

    out = pl.pallas_call(
        _stats_kernel,
        grid=(2,),
        in_specs=[pl.BlockSpec((1, NPAIR // 2, FLAT // 128, 128),
                               lambda g: (g, 0, 0, 0))],
        out_specs=pl.BlockSpec((1, NPAIR // 2, 128), lambda g: (g, 0, 0)),
        out_shape=jax.ShapeDtypeStruct((2, NPAIR // 2, 128), jnp.float32),
        compiler_params=pltpu.CompilerParams(
            dimension_semantics=("parallel",)),
    )(r)
    stds = out[:, :, 0].reshape(NPAIR)
    return jnp.mean(jnp.maximum(stds[0:8], stds[8:16]))


# speedup vs baseline: 1.3066x; 1.3066x over previous
import numpy as np
import jax
import jax.numpy as jnp
from jax.experimental import pallas as pl
from jax.experimental.pallas import tpu as pltpu

N = 4096
D = 3
TR = 1024
NPAIR = 16
NT = N // TR
FLAT = N * D

_POS_LO = np.float32(0.15) * np.float32(FLAT - 1)
_POS_HI = np.float32(0.85) * np.float32(FLAT - 1)
RANK_LO = int(np.floor(_POS_LO))
RANK_HI = int(np.floor(_POS_HI))
FRAC_LO = np.float32(_POS_LO - np.floor(_POS_LO))
FRAC_HI = np.float32(_POS_HI - np.floor(_POS_HI))

INT_MIN = np.int32(-2**31)
INT_MAX = np.int32(2**31 - 1)


def _residual_kernel(s1_ref, s2t_ref, out_ref):
    s1 = s1_ref[0]
    s2t = s2t_ref[0]
    b0 = s2t[0:1, :]
    b1 = s2t[1:2, :]
    b2 = s2t[2:3, :]
    a0 = s1[:, 0:1]
    a1 = s1[:, 1:2]
    a2 = s1[:, 2:3]
    s2sq = (b0 * b0 + b1 * b1) + b2 * b2
    hi = s2sq.astype(jnp.bfloat16)
    r1 = s2sq - hi.astype(jnp.float32)
    mid = r1.astype(jnp.bfloat16)
    lo = (r1 - mid.astype(jnp.float32)).astype(jnp.bfloat16)
    w = jnp.concatenate(
        [(-(s2t + s2t)).astype(jnp.bfloat16), hi, mid, lo], axis=0)
    ones = jnp.ones((TR, 3), dtype=jnp.bfloat16)
    s1aug = jnp.concatenate([s1.astype(jnp.bfloat16), ones], axis=1)
    scores = jnp.dot(s1aug, w, preferred_element_type=jnp.float32)
    mins = jnp.min(scores, axis=1, keepdims=True)
    ismin = scores == mins
    big = jnp.float32(2.0)
    c0 = jnp.min(jnp.where(ismin, b0, big), axis=1, keepdims=True)
    c1 = jnp.min(jnp.where(ismin, b1, big), axis=1, keepdims=True)
    c2 = jnp.min(jnp.where(ismin, b2, big), axis=1, keepdims=True)
    out_ref[0] = jnp.concatenate([a0 - c0, a1 - c1, a2 - c2], axis=1)


def _to_ordered(i):
    neg = jnp.bitwise_xor(jnp.bitwise_not(i), INT_MIN)
    return jnp.where(i >= 0, i, neg)


def _from_ordered(k):
    i = jnp.where(k >= 0, k, jnp.bitwise_not(jnp.bitwise_xor(k, INT_MIN)))
    return jax.lax.bitcast_convert_type(i, jnp.float32)


def _count_lt(keys, t):
    return jnp.sum((keys < t).astype(jnp.int32), axis=(1, 2), keepdims=True)


def _order_stat(keys, rank):
    def body(b, a):
        bit = jnp.left_shift(jnp.int32(1), 31 - b)
        t = a + bit
        cnt = _count_lt(keys, t)
        return jnp.where(cnt <= rank, t, a)
    a0 = jnp.full((keys.shape[0], 1, 1), INT_MIN, dtype=jnp.int32)
    return jax.lax.fori_loop(0, 32, body, a0)


def _next_order_stat(keys, v, rank):
    c_le = jnp.sum((keys <= v).astype(jnp.int32), axis=(1, 2), keepdims=True)
    above = jnp.where(keys > v, keys, INT_MAX)
    nxt = jnp.min(above, axis=(1, 2), keepdims=True)
    return jnp.where(c_le >= rank + 2, v, nxt)


def _masked_std1(r, mask):
    zero = jnp.float32(0.0)
    nm = jnp.sum(jnp.where(mask, jnp.float32(1.0), zero), axis=(1, 2),
                 keepdims=True)
    tot = jnp.sum(jnp.where(mask, r, zero), axis=(1, 2), keepdims=True)
    mean = tot / nm
    sq = jnp.sum(jnp.where(mask, (r - mean) ** 2, zero), axis=(1, 2),
                 keepdims=True)
    return jnp.sqrt(sq / (nm - 1.0)), nm


def _stats_kernel(r_ref, out_ref):
    r = r_ref[0]
    i = jax.lax.bitcast_convert_type(r, jnp.int32)
    keys = _to_ordered(i)

    v_lo0 = _order_stat(keys, RANK_LO)
    v_lo1 = _next_order_stat(keys, v_lo0, RANK_LO)
    v_hi0 = _order_stat(keys, RANK_HI)
    v_hi1 = _next_order_stat(keys, v_hi0, RANK_HI)

    q0 = _from_ordered(v_lo0) * (1.0 - FRAC_LO) + _from_ordered(v_lo1) * FRAC_LO
    q1 = _from_ordered(v_hi0) * (1.0 - FRAC_HI) + _from_ordered(v_hi1) * FRAC_HI

    mask = (r < q0) | (r > q1)
    std_m, nm = _masked_std1(r, mask)
    all_false = nm == 0.0

    nf = jnp.float32(FLAT)
    mean_a = jnp.sum(r, axis=(1, 2), keepdims=True) / nf
    ss = jnp.sum((r - mean_a) ** 2, axis=(1, 2), keepdims=True)
    std_a = jnp.sqrt(ss / (nf - 1.0))
    mask2 = jnp.abs(r - mean_a) > std_a
    std_m2, nm2 = _masked_std1(r, mask2)
    fb = jnp.where(nm2 == 0.0, std_a, std_m2)

    stds = jnp.where(all_false, fb, std_m)
    out_ref[0] = jnp.broadcast_to(stds.reshape(8, 1), (8, 128))


def kernel(x, y):
    s1 = jnp.concatenate([x, y], axis=0)
    s2 = jnp.concatenate([y, x], axis=0)
    s2t = s2.transpose(0, 2, 1)

    resid = pl.pallas_call(
        _residual_kernel,
        grid=(NPAIR, NT),
        in_specs=[
            pl.BlockSpec((1, TR, D), lambda b, t: (b, t, 0)),
            pl.BlockSpec((1, D, N), lambda b, t: (b, 0, 0)),
        ],
        out_specs=pl.BlockSpec((1, TR, D), lambda b, t: (b, t, 0)),
        out_shape=jax.ShapeDtypeStruct((NPAIR, N, D), jnp.float32),
        compiler_params=pltpu.CompilerParams(
            dimension_semantics=("parallel", "arbitrary")),
    )(s1, s2t)

    r = resid.reshape(2, NPAIR // 2, FLAT // 128, 128)

    out = pl.pallas_call(
        _stats_kernel,
        grid=(2,),
        in_specs=[pl.BlockSpec((1, NPAIR // 2, FLAT // 128, 128),
                               lambda g: (g, 0, 0, 0))],
        out_specs=pl.BlockSpec((1, NPAIR // 2, 128), lambda g: (g, 0, 0)),
        out_shape=jax.ShapeDtypeStruct((2, NPAIR // 2, 128), jnp.float32),
        compiler_params=pltpu.CompilerParams(
            dimension_semantics=("parallel",)),
    )(r)
    stds = out[:, :, 0].reshape(NPAIR)
    return jnp.mean(jnp.maximum(stds[0:8], stds[8:16]))


# speedup vs baseline: 1.3285x; 1.0167x over previous
import numpy as np
import jax
import jax.numpy as jnp
from jax.experimental import pallas as pl
from jax.experimental.pallas import tpu as pltpu

N = 4096
D = 3
TR = 2048
NPAIR = 16
NT = N // TR
FLAT = N * D

_POS_LO = np.float32(0.15) * np.float32(FLAT - 1)
_POS_HI = np.float32(0.85) * np.float32(FLAT - 1)
RANK_LO = int(np.floor(_POS_LO))
RANK_HI = int(np.floor(_POS_HI))
FRAC_LO = np.float32(_POS_LO - np.floor(_POS_LO))
FRAC_HI = np.float32(_POS_HI - np.floor(_POS_HI))

INT_MIN = np.int32(-2**31)
INT_MAX = np.int32(2**31 - 1)


def _residual_kernel(s1_ref, s2t_ref, out_ref):
    s1 = s1_ref[0]
    s2t = s2t_ref[0]
    b0 = s2t[0:1, :]
    b1 = s2t[1:2, :]
    b2 = s2t[2:3, :]
    a0 = s1[:, 0:1]
    a1 = s1[:, 1:2]
    a2 = s1[:, 2:3]
    s2sq = (b0 * b0 + b1 * b1) + b2 * b2
    hi = s2sq.astype(jnp.bfloat16)
    r1 = s2sq - hi.astype(jnp.float32)
    mid = r1.astype(jnp.bfloat16)
    lo = (r1 - mid.astype(jnp.float32)).astype(jnp.bfloat16)
    w = jnp.concatenate(
        [(-(s2t + s2t)).astype(jnp.bfloat16), hi, mid, lo], axis=0)
    ones = jnp.ones((TR, 3), dtype=jnp.bfloat16)
    s1aug = jnp.concatenate([s1.astype(jnp.bfloat16), ones], axis=1)
    scores = jnp.dot(s1aug, w, preferred_element_type=jnp.float32)
    mins = jnp.min(scores, axis=1, keepdims=True)
    ismin = scores == mins
    big = jnp.float32(2.0)
    c0 = jnp.min(jnp.where(ismin, b0, big), axis=1, keepdims=True)
    c1 = jnp.min(jnp.where(ismin, b1, big), axis=1, keepdims=True)
    c2 = jnp.min(jnp.where(ismin, b2, big), axis=1, keepdims=True)
    out_ref[0] = jnp.concatenate([a0 - c0, a1 - c1, a2 - c2], axis=1)


def _to_ordered(i):
    neg = jnp.bitwise_xor(jnp.bitwise_not(i), INT_MIN)
    return jnp.where(i >= 0, i, neg)


def _from_ordered(k):
    i = jnp.where(k >= 0, k, jnp.bitwise_not(jnp.bitwise_xor(k, INT_MIN)))
    return jax.lax.bitcast_convert_type(i, jnp.float32)


def _count_lt(keys, t):
    return jnp.sum((keys < t).astype(jnp.int32), axis=(1, 2), keepdims=True)


def _order_stat(keys, rank):
    def body(b, a):
        bit = jnp.left_shift(jnp.int32(1), 31 - b)
        t = a + bit
        cnt = _count_lt(keys, t)
        return jnp.where(cnt <= rank, t, a)
    a0 = jnp.full((keys.shape[0], 1, 1), INT_MIN, dtype=jnp.int32)
    return jax.lax.fori_loop(0, 32, body, a0)


def _next_order_stat(keys, v, rank):
    c_le = jnp.sum((keys <= v).astype(jnp.int32), axis=(1, 2), keepdims=True)
    above = jnp.where(keys > v, keys, INT_MAX)
    nxt = jnp.min(above, axis=(1, 2), keepdims=True)
    return jnp.where(c_le >= rank + 2, v, nxt)


def _masked_std1(r, mask):
    zero = jnp.float32(0.0)
    nm = jnp.sum(jnp.where(mask, jnp.float32(1.0), zero), axis=(1, 2),
                 keepdims=True)
    tot = jnp.sum(jnp.where(mask, r, zero), axis=(1, 2), keepdims=True)
    mean = tot / nm
    sq = jnp.sum(jnp.where(mask, (r - mean) ** 2, zero), axis=(1, 2),
                 keepdims=True)
    return jnp.sqrt(sq / (nm - 1.0)), nm


def _stats_kernel(r_ref, out_ref):
    r = r_ref[0]
    i = jax.lax.bitcast_convert_type(r, jnp.int32)
    keys = _to_ordered(i)

    v_lo0 = _order_stat(keys, RANK_LO)
    v_lo1 = _next_order_stat(keys, v_lo0, RANK_LO)
    v_hi0 = _order_stat(keys, RANK_HI)
    v_hi1 = _next_order_stat(keys, v_hi0, RANK_HI)

    q0 = _from_ordered(v_lo0) * (1.0 - FRAC_LO) + _from_ordered(v_lo1) * FRAC_LO
    q1 = _from_ordered(v_hi0) * (1.0 - FRAC_HI) + _from_ordered(v_hi1) * FRAC_HI

    mask = (r < q0) | (r > q1)
    std_m, nm = _masked_std1(r, mask)
    all_false = nm == 0.0

    nf = jnp.float32(FLAT)
    mean_a = jnp.sum(r, axis=(1, 2), keepdims=True) / nf
    ss = jnp.sum((r - mean_a) ** 2, axis=(1, 2), keepdims=True)
    std_a = jnp.sqrt(ss / (nf - 1.0))
    mask2 = jnp.abs(r - mean_a) > std_a
    std_m2, nm2 = _masked_std1(r, mask2)
    fb = jnp.where(nm2 == 0.0, std_a, std_m2)

    stds = jnp.where(all_false, fb, std_m)
    out_ref[0] = jnp.broadcast_to(stds.reshape(8, 1), (8, 128))


def kernel(x, y):
    s1 = jnp.concatenate([x, y], axis=0)
    s2 = jnp.concatenate([y, x], axis=0)
    s2t = s2.transpose(0, 2, 1)

    resid = pl.pallas_call(
        _residual_kernel,
        grid=(NPAIR, NT),
        in_specs=[
            pl.BlockSpec((1, TR, D), lambda b, t: (b, t, 0)),
            pl.BlockSpec((1, D, N), lambda b, t: (b, 0, 0)),
        ],
        out_specs=pl.BlockSpec((1, TR, D), lambda b, t: (b, t, 0)),
        out_shape=jax.ShapeDtypeStruct((NPAIR, N, D), jnp.float32),
        compiler_params=pltpu.CompilerParams(
            dimension_semantics=("parallel", "arbitrary")),
    )(s1, s2t)

    r = resid.reshape(2, NPAIR // 2, FLAT // 128, 128)

    out = pl.pallas_call(
        _stats_kernel,
        grid=(2,),
        in_specs=[pl.BlockSpec((1, NPAIR // 2, FLAT // 128, 128),
                               lambda g: (g, 0, 0, 0))],
        out_specs=pl.BlockSpec((1, NPAIR // 2, 128), lambda g: (g, 0, 0)),
        out_shape=jax.ShapeDtypeStruct((2, NPAIR // 2, 128), jnp.float32),
        compiler_params=pltpu.CompilerParams(
            dimension_semantics=("parallel",)),
    )(r)
    stds = out[:, :, 0].reshape(NPAIR)
    return jnp.mean(jnp.maximum(stds[0:8], stds[8:16]))


# speedup vs baseline: 1.3360x; 1.0056x over previous
import numpy as np
import jax
import jax.numpy as jnp
from jax.experimental import pallas as pl
from jax.experimental.pallas import tpu as pltpu

N = 4096
D = 3
TR = 2048
NPAIR = 16
NT = N // TR
FLAT = N * D

_POS_LO = np.float32(0.15) * np.float32(FLAT - 1)
_POS_HI = np.float32(0.85) * np.float32(FLAT - 1)
RANK_LO = int(np.floor(_POS_LO))
RANK_HI = int(np.floor(_POS_HI))
FRAC_LO = np.float32(_POS_LO - np.floor(_POS_LO))
FRAC_HI = np.float32(_POS_HI - np.floor(_POS_HI))

INT_MIN = np.int32(-2**31)
INT_MAX = np.int32(2**31 - 1)


def _residual_kernel(xt_ref, yt_ref, xft_ref, yft_ref, out_ref):
    rev = pl.program_id(0) >= 8
    s1 = jnp.where(rev, yt_ref[0], xt_ref[0])
    s2t = jnp.where(rev, xft_ref[0], yft_ref[0])
    b0 = s2t[0:1, :]
    b1 = s2t[1:2, :]
    b2 = s2t[2:3, :]
    a0 = s1[:, 0:1]
    a1 = s1[:, 1:2]
    a2 = s1[:, 2:3]
    s2sq = (b0 * b0 + b1 * b1) + b2 * b2
    hi = s2sq.astype(jnp.bfloat16)
    r1 = s2sq - hi.astype(jnp.float32)
    mid = r1.astype(jnp.bfloat16)
    lo = (r1 - mid.astype(jnp.float32)).astype(jnp.bfloat16)
    w = jnp.concatenate(
        [(-(s2t + s2t)).astype(jnp.bfloat16), hi, mid, lo], axis=0)
    ones = jnp.ones((TR, 3), dtype=jnp.bfloat16)
    s1aug = jnp.concatenate([s1.astype(jnp.bfloat16), ones], axis=1)
    scores = jnp.dot(s1aug, w, preferred_element_type=jnp.float32)
    mins = jnp.min(scores, axis=1, keepdims=True)
    ismin = scores == mins
    big = jnp.float32(2.0)
    c0 = jnp.min(jnp.where(ismin, b0, big), axis=1, keepdims=True)
    c1 = jnp.min(jnp.where(ismin, b1, big), axis=1, keepdims=True)
    c2 = jnp.min(jnp.where(ismin, b2, big), axis=1, keepdims=True)
    out_ref[0] = jnp.concatenate([a0 - c0, a1 - c1, a2 - c2], axis=1)


def _to_ordered(i):
    neg = jnp.bitwise_xor(jnp.bitwise_not(i), INT_MIN)
    return jnp.where(i >= 0, i, neg)


def _from_ordered(k):
    i = jnp.where(k >= 0, k, jnp.bitwise_not(jnp.bitwise_xor(k, INT_MIN)))
    return jax.lax.bitcast_convert_type(i, jnp.float32)


def _count_lt(keys, t):
    return jnp.sum((keys < t).astype(jnp.int32), axis=(1, 2), keepdims=True)


def _order_stat(keys, rank):
    def body(b, a):
        bit = jnp.left_shift(jnp.int32(1), 31 - b)
        t = a + bit
        cnt = _count_lt(keys, t)
        return jnp.where(cnt <= rank, t, a)
    a0 = jnp.full((keys.shape[0], 1, 1), INT_MIN, dtype=jnp.int32)
    return jax.lax.fori_loop(0, 32, body, a0)


def _next_order_stat(keys, v, rank):
    c_le = jnp.sum((keys <= v).astype(jnp.int32), axis=(1, 2), keepdims=True)
    above = jnp.where(keys > v, keys, INT_MAX)
    nxt = jnp.min(above, axis=(1, 2), keepdims=True)
    return jnp.where(c_le >= rank + 2, v, nxt)


def _masked_std1(r, mask):
    zero = jnp.float32(0.0)
    nm = jnp.sum(jnp.where(mask, jnp.float32(1.0), zero), axis=(1, 2),
                 keepdims=True)
    tot = jnp.sum(jnp.where(mask, r, zero), axis=(1, 2), keepdims=True)
    mean = tot / nm
    sq = jnp.sum(jnp.where(mask, (r - mean) ** 2, zero), axis=(1, 2),
                 keepdims=True)
    return jnp.sqrt(sq / (nm - 1.0)), nm


def _stats_kernel(r_ref, out_ref):
    r = r_ref[0]
    i = jax.lax.bitcast_convert_type(r, jnp.int32)
    keys = _to_ordered(i)

    v_lo0 = _order_stat(keys, RANK_LO)
    v_lo1 = _next_order_stat(keys, v_lo0, RANK_LO)
    v_hi0 = _order_stat(keys, RANK_HI)
    v_hi1 = _next_order_stat(keys, v_hi0, RANK_HI)

    q0 = _from_ordered(v_lo0) * (1.0 - FRAC_LO) + _from_ordered(v_lo1) * FRAC_LO
    q1 = _from_ordered(v_hi0) * (1.0 - FRAC_HI) + _from_ordered(v_hi1) * FRAC_HI

    mask = (r < q0) | (r > q1)
    std_m, nm = _masked_std1(r, mask)
    all_false = nm == 0.0

    nf = jnp.float32(FLAT)
    mean_a = jnp.sum(r, axis=(1, 2), keepdims=True) / nf
    ss = jnp.sum((r - mean_a) ** 2, axis=(1, 2), keepdims=True)
    std_a = jnp.sqrt(ss / (nf - 1.0))
    mask2 = jnp.abs(r - mean_a) > std_a
    std_m2, nm2 = _masked_std1(r, mask2)
    fb = jnp.where(nm2 == 0.0, std_a, std_m2)

    stds = jnp.where(all_false, fb, std_m)
    out_ref[0] = jnp.broadcast_to(stds.reshape(8, 1), (8, 128))


def kernel(x, y):
    xt = x.transpose(0, 2, 1)
    yt = y.transpose(0, 2, 1)

    resid = pl.pallas_call(
        _residual_kernel,
        grid=(NPAIR, NT),
        in_specs=[
            pl.BlockSpec((1, TR, D), lambda b, t: (b % 8, t, 0)),
            pl.BlockSpec((1, TR, D), lambda b, t: (b % 8, t, 0)),
            pl.BlockSpec((1, D, N), lambda b, t: (b % 8, 0, 0)),
            pl.BlockSpec((1, D, N), lambda b, t: (b % 8, 0, 0)),
        ],
        out_specs=pl.BlockSpec((1, TR, D), lambda b, t: (b, t, 0)),
        out_shape=jax.ShapeDtypeStruct((NPAIR, N, D), jnp.float32),
        compiler_params=pltpu.CompilerParams(
            dimension_semantics=("parallel", "arbitrary")),
    )(x, y, xt, yt)

    r = resid.reshape(2, NPAIR // 2, FLAT // 128, 128)

    out = pl.pallas_call(
        _stats_kernel,
        grid=(2,),
        in_specs=[pl.BlockSpec((1, NPAIR // 2, FLAT // 128, 128),
                               lambda g: (g, 0, 0, 0))],
        out_specs=pl.BlockSpec((1, NPAIR // 2, 128), lambda g: (g, 0, 0)),
        out_shape=jax.ShapeDtypeStruct((2, NPAIR // 2, 128), jnp.float32),
        compiler_params=pltpu.CompilerParams(
            dimension_semantics=("parallel",)),
    )(r)
    stds = out[:, :, 0].reshape(NPAIR)
    return jnp.mean(jnp.maximum(stds[0:8], stds[8:16]))
